# baseline (speedup 1.0000x reference)
import functools

import jax
import jax.numpy as jnp
from jax import lax
from jax.experimental import pallas as pl
from jax.experimental.pallas import tpu as pltpu
from jax.experimental.pallas import tpu_sc as plsc

NBUF = 2
LANES = 128


def _make_gather(batch: int, seq: int, d: int, vocab: int):
    info = plsc.get_sparse_core_info()
    nc, ns, nl = info.num_cores, info.num_subcores, info.num_lanes
    nw = nc * ns
    assert batch % nw == 0
    bpw = batch // nw
    toks_per_w = bpw * seq

    mesh = plsc.VectorSubcoreMesh(core_axis_name="c", subcore_axis_name="s")

    @functools.partial(
        pl.kernel,
        out_type=jax.ShapeDtypeStruct((seq, d, batch), jnp.float32),
        mesh=mesh,
        scratch_types=[
            pltpu.VMEM((toks_per_w,), jnp.int32),
            pltpu.VMEM((seq, bpw), jnp.int32),
            pltpu.VMEM((NBUF, bpw, LANES), jnp.float32),
            pltpu.VMEM((NBUF, d, bpw), jnp.float32),
        ]
        + [pltpu.SemaphoreType.DMA] * (2 * NBUF + 1),
        compiler_params=pltpu.CompilerParams(needs_layout_passes=False),
    )
    def gather_kernel(idx_hbm, table_hbm, out_hbm, idx_v, sidx_v, gbuf, tbuf, *sems):
        gsems = sems[:NBUF]
        osems = sems[NBUF : 2 * NBUF]
        isem = sems[2 * NBUF]
        wid = lax.axis_index("s") * nc + lax.axis_index("c")
        base = wid * bpw
        pltpu.sync_copy(idx_hbm.at[pl.ds(base * seq, toks_per_w)], idx_v)

        lanes = lax.iota(jnp.int32, nl)
        stride = lanes * seq

        def build_s(s, carry):
            for jb in range(bpw // nl):
                v = plsc.load_gather(idx_v, [stride + (jb * nl * seq + s)])
                sidx_v[s, pl.ds(jb * nl, nl)] = v
            return carry

        lax.fori_loop(0, seq, build_s, 0)

        def gather_copy(s, buf):
            return pltpu.make_async_copy(
                table_hbm.at[sidx_v.at[s]], gbuf.at[buf], gsems[buf]
            )

        def out_copy(s, buf):
            return pltpu.make_async_copy(
                tbuf.at[buf],
                out_hbm.at[s, :, pl.ds(base, bpw)],
                osems[buf],
            )

        for b in range(NBUF):
            gather_copy(b, b).start()

        zero = lax.iota(jnp.int32, nl) * 0

        def body(g, carry):
            for b in range(NBUF):
                s = NBUF * g + b
                gather_copy(s, b).wait()

                @pl.when(s >= NBUF)
                def _():
                    out_copy(s - NBUF, b).wait()

                for jb in range(bpw // nl):
                    rows = lanes + (jb * nl)
                    for dd in range(d):
                        v = plsc.load_gather(gbuf.at[b], [rows, zero + dd])
                        tbuf[b, dd, pl.ds(jb * nl, nl)] = v

                @pl.when(s + NBUF < seq)
                def _():
                    gather_copy(s + NBUF, b).start()

                out_copy(s, b).start()

            return carry

        lax.fori_loop(0, seq // NBUF, body, 0)

        for b in range(NBUF):
            out_copy(seq - NBUF + b, b).wait()
        _ = isem

    return gather_kernel


def kernel(x, table):
    b, s = x.shape
    v, d = table.shape
    idx = x.reshape(-1).astype(jnp.int32)
    table_p = jnp.pad(table, ((0, 0), (0, LANES - d)))
    out_t = _make_gather(b, s, d, v)(idx, table_p)
    return out_t.transpose(2, 0, 1)


# speedup vs baseline: 1.9318x; 1.9318x over previous
import functools

import jax
import jax.numpy as jnp
from jax import lax
from jax.experimental import pallas as pl
from jax.experimental.pallas import tpu as pltpu
from jax.experimental.pallas import tpu_sc as plsc

NBUF = 4
LANES = 128
D_VALID = 64


def _make_gather(batch: int, seq: int, vocab: int):
    info = plsc.get_sparse_core_info()
    nc, ns = info.num_cores, info.num_subcores
    nw = nc * ns
    assert batch % nw == 0
    rows_per_w = batch // nw
    toks_per_w = rows_per_w * seq

    mesh = plsc.VectorSubcoreMesh(core_axis_name="c", subcore_axis_name="s")

    @functools.partial(
        pl.kernel,
        out_type=jax.ShapeDtypeStruct((batch * seq, LANES), jnp.float32),
        mesh=mesh,
        scratch_types=[
            pltpu.VMEM((toks_per_w,), jnp.int32),
            pltpu.VMEM((NBUF, seq, LANES), jnp.float32),
        ]
        + [pltpu.SemaphoreType.DMA] * NBUF,
    )
    def gather_kernel(idx_hbm, table_hbm, out_hbm, idx_v, rows_v, *sems):
        wid = lax.axis_index("s") * nc + lax.axis_index("c")
        base = wid * rows_per_w
        pltpu.sync_copy(idx_hbm.at[pl.ds(base * seq, toks_per_w)], idx_v)

        def gather_copy(i, buf):
            return pltpu.make_async_copy(
                table_hbm.at[idx_v.at[pl.ds(i * seq, seq)]],
                rows_v.at[buf],
                sems[buf],
            )

        for b in range(NBUF):
            gather_copy(b, b).start()

        def body(g, carry):
            for b in range(NBUF):
                i = NBUF * g + b
                gather_copy(i, b).wait()
                pltpu.sync_copy(
                    rows_v.at[b], out_hbm.at[pl.ds((base + i) * seq, seq)]
                )

                @pl.when(i + NBUF < rows_per_w)
                def _():
                    gather_copy(i + NBUF, b).start()

            return carry

        lax.fori_loop(0, rows_per_w // NBUF, body, 0)

    return gather_kernel


def kernel(x, table):
    b, s = x.shape
    v, d = table.shape
    idx = x.reshape(-1).astype(jnp.int32)
    table_p = jnp.pad(table, ((0, 0), (0, LANES - d)))
    out_p = _make_gather(b, s, v)(idx, table_p)
    return out_p[:, :d].reshape(b, s, d)


# speedup vs baseline: 1.9367x; 1.0025x over previous
import functools

import jax
import jax.numpy as jnp
from jax import lax
from jax.experimental import pallas as pl
from jax.experimental.pallas import tpu as pltpu
from jax.experimental.pallas import tpu_sc as plsc

NBUF = 4
LANES = 128
D_VALID = 64


def _make_gather(batch: int, seq: int, vocab: int):
    info = plsc.get_sparse_core_info()
    nc, ns = info.num_cores, info.num_subcores
    nw = nc * ns
    assert batch % nw == 0
    rows_per_w = batch // nw
    toks_per_w = rows_per_w * seq

    mesh = plsc.VectorSubcoreMesh(core_axis_name="c", subcore_axis_name="s")

    @functools.partial(
        pl.kernel,
        out_type=jax.ShapeDtypeStruct((batch * seq, LANES), jnp.float32),
        mesh=mesh,
        scratch_types=[
            pltpu.VMEM((toks_per_w,), jnp.int32),
            pltpu.VMEM((NBUF, seq, LANES), jnp.float32),
        ]
        + [pltpu.SemaphoreType.DMA] * NBUF,
        compiler_params=pltpu.CompilerParams(disable_bounds_checks=True),
    )
    def gather_kernel(idx_hbm, table_hbm, out_hbm, idx_v, rows_v, *sems):
        wid = lax.axis_index("s") * nc + lax.axis_index("c")
        base = wid * rows_per_w
        pltpu.sync_copy(idx_hbm.at[pl.ds(base * seq, toks_per_w)], idx_v)

        def gather_copy(i, buf):
            return pltpu.make_async_copy(
                table_hbm.at[idx_v.at[pl.ds(i * seq, seq)]],
                rows_v.at[buf],
                sems[buf],
            )

        for b in range(NBUF):
            gather_copy(b, b).start()

        def body(g, carry):
            for b in range(NBUF):
                i = NBUF * g + b
                gather_copy(i, b).wait()
                pltpu.sync_copy(
                    rows_v.at[b], out_hbm.at[pl.ds((base + i) * seq, seq)]
                )

                @pl.when(i + NBUF < rows_per_w)
                def _():
                    gather_copy(i + NBUF, b).start()

            return carry

        lax.fori_loop(0, rows_per_w // NBUF, body, 0)

    return gather_kernel


def kernel(x, table):
    b, s = x.shape
    v, d = table.shape
    idx = x.reshape(-1).astype(jnp.int32)
    table_p = jnp.pad(table, ((0, 0), (0, LANES - d)))
    out_p = _make_gather(b, s, v)(idx, table_p)
    return out_p[:, :d].reshape(b, s, d)
